# baseline (speedup 1.0000x reference)
import jax
import jax.numpy as jnp
from jax.experimental import pallas as pl

N = 50000
D = 512
RATIO = 0.5
K = max(int(N * RATIO), 1)


def _matvec_block(x_ref, w_ref, o_ref):
    o_ref[...] = jnp.dot(x_ref[...], w_ref[...],
                         preferred_element_type=jnp.float32)


def _scores_matvec(x, W):
    BLK = 2000
    grid = (N // BLK,)
    out = pl.pallas_call(
        _matvec_block,
        grid=grid,
        in_specs=[
            pl.BlockSpec((BLK, D), lambda i: (i, 0)),
            pl.BlockSpec((D, 1), lambda i: (0, 0)),
        ],
        out_specs=pl.BlockSpec((BLK, 1), lambda i: (i, 0)),
        out_shape=jax.ShapeDtypeStruct((N, 1), jnp.float32),
    )(x, W)
    return out[:, 0]


def kernel(x, edge_index, batch, W):
    scores = _scores_matvec(x, W)
    edge_index_i = edge_index[0]
    edge_index_j = edge_index[1]
    neighbor_scores = scores[edge_index_j]
    aggregated = jnp.zeros_like(scores).at[edge_index_i].add(neighbor_scores)
    scores = scores + aggregated
    _, perm = jax.lax.top_k(scores, K)
    x_pool = x[perm]
    mask = jnp.ones((N,), dtype=bool).at[perm].set(False)
    (keep_idx,) = jnp.nonzero(mask, size=N - K)
    new_edge_index_i = edge_index_i[keep_idx]
    new_edge_index_j = edge_index_j[keep_idx]
    new_mask = jnp.zeros((N,), dtype=jnp.int32).at[perm].set(
        jnp.arange(K, dtype=jnp.int32)
    )
    new_edge_index_i = new_mask[new_edge_index_i]
    new_edge_index_j = new_mask[new_edge_index_j]
    batch_pool = batch[perm]
    edge_index_pool = jnp.stack([new_edge_index_i, new_edge_index_j], axis=0)
    return (x_pool, edge_index_pool, perm, batch_pool, scores[perm])
